# Initial kernel scaffold; baseline (speedup 1.0000x reference)
import jax
import jax.numpy as jnp
from jax.experimental import pallas as pl


def kernel(inputs, connectivity, weights, bias):
    raise NotImplementedError("write your pallas kernel here")


# speedup vs baseline: 5.0419x; 5.0419x over previous
import functools

import jax
import jax.numpy as jnp
from jax import lax
from jax.experimental import pallas as pl
from jax.experimental.pallas import tpu as pltpu
from jax.experimental.pallas import tpu_sc as plsc

IN_F = 16384
OUT_F = 16384
NNZ = 268435
B = 64

NC = 2
NS = 16
NW = NC * NS
L = 16

K = 128
NBLK = -(-NNZ // (NW * K))
CPT = NBLK * K
NNZ_PAD = CPT * NW

_mesh = plsc.VectorSubcoreMesh(core_axis_name="c", subcore_axis_name="s")

_BCAST_DNUMS = lax.GatherDimensionNumbers(
    offset_dims=(), collapsed_slice_dims=(0,), start_index_map=(0,))


def _bcast_lane(vec, j):
    idx = jnp.full((L, 1), j, jnp.int32)
    return lax.gather(vec, idx, _BCAST_DNUMS, (1,),
                      mode=lax.GatherScatterMode.PROMISE_IN_BOUNDS)


@functools.partial(
    pl.kernel,
    mesh=_mesh,
    compiler_params=pltpu.CompilerParams(use_tc_tiling_on_sc=False),
    out_type=jax.ShapeDtypeStruct((NC, OUT_F, B), jnp.float32),
    scratch_types=[
        pltpu.VMEM((K,), jnp.int32),
        pltpu.VMEM((K,), jnp.int32),
        pltpu.VMEM((K,), jnp.float32),
        pltpu.VMEM((K, B), jnp.float32),
        pltpu.VMEM_SHARED((OUT_F, B), jnp.float32),
        pltpu.SemaphoreType.DMA,
    ],
)
def _sc_spmm(xt_hbm, cols_hbm, rows_hbm, w_hbm, out_hbm,
             cols_v, rows_v, w_v, xrows_v, acc, sem):
    cid = lax.axis_index("c")
    sid = lax.axis_index("s")
    wid = sid * NC + cid

    zero16 = jnp.zeros((L,), jnp.float32)

    def zbody(i, _):
        xrows_v[i // 4, pl.ds((i % 4) * L, L)] = zero16
        return 0

    lax.fori_loop(0, K * 4, zbody, 0)
    rows_per_tile = OUT_F // NS
    for j in range(rows_per_tile // K):
        pltpu.sync_copy(xrows_v, acc.at[pl.ds(sid * rows_per_tile + j * K, K)])
    plsc.subcore_barrier()

    base = wid * CPT

    def block_body(b, _):
        off = base + b * K
        pltpu.sync_copy(cols_hbm.at[pl.ds(off, K)], cols_v)
        pltpu.sync_copy(rows_hbm.at[pl.ds(off, K)], rows_v)
        pltpu.sync_copy(w_hbm.at[pl.ds(off, K)], w_v)
        pltpu.async_copy(xt_hbm.at[cols_v], xrows_v, sem).wait()

        def mul_body(i16, _):
            w16 = w_v[pl.ds(i16 * L, L)]
            for j in range(L):
                wb = _bcast_lane(w16, j)
                e = i16 * L + j
                for v in range(B // L):
                    xrows_v[e, pl.ds(v * L, L)] = xrows_v[e, pl.ds(v * L, L)] * wb
            return 0

        lax.fori_loop(0, K // L, mul_body, 0)
        pltpu.sync_copy(xrows_v, acc.at[rows_v], add=True)
        return 0

    lax.fori_loop(0, NBLK, block_body, 0)

    plsc.subcore_barrier()
    pltpu.sync_copy(acc.at[pl.ds(sid * rows_per_tile, rows_per_tile)],
                    out_hbm.at[cid, pl.ds(sid * rows_per_tile, rows_per_tile)])


_RB = 1024


def _combine_body(p_ref, b_ref, o_ref):
    s = p_ref[0] + p_ref[1]
    o_ref[...] = s.T + b_ref[...]


def kernel(inputs, connectivity, weights, bias):
    lead = inputs.shape[:-1]
    x = inputs.reshape(-1, inputs.shape[-1])
    xt = x.T

    rows = connectivity[0]
    cols = connectivity[1]
    pad = NNZ_PAD - NNZ
    cols_p = jnp.concatenate([cols, jnp.zeros((pad,), jnp.int32)])
    rows_p = jnp.concatenate([rows, jnp.zeros((pad,), jnp.int32)])
    w_p = jnp.concatenate([weights, jnp.zeros((pad,), jnp.float32)])

    partial = _sc_spmm(xt, cols_p, rows_p, w_p)

    out = pl.pallas_call(
        _combine_body,
        grid=(OUT_F // _RB,),
        in_specs=[
            pl.BlockSpec((NC, _RB, B), lambda i: (0, i, 0)),
            pl.BlockSpec((1, _RB), lambda i: (0, i)),
        ],
        out_specs=pl.BlockSpec((B, _RB), lambda i: (0, i)),
        out_shape=jax.ShapeDtypeStruct((B, OUT_F), jnp.float32),
    )(partial, bias.reshape(1, OUT_F))
    return out.reshape((*lead, OUT_F))


# speedup vs baseline: 12.3858x; 2.4566x over previous
import functools

import jax
import jax.numpy as jnp
from jax import lax
from jax.experimental import pallas as pl
from jax.experimental.pallas import tpu as pltpu
from jax.experimental.pallas import tpu_sc as plsc

IN_F = 16384
OUT_F = 16384
NNZ = 268435
B = 64

NC = 2
NS = 16
NW = NC * NS
L = 16

K = 128
NBLK = -(-NNZ // (NW * K))
CPT = NBLK * K
NNZ_PAD = CPT * NW

_mesh = plsc.VectorSubcoreMesh(core_axis_name="c", subcore_axis_name="s")

_BCAST_DNUMS = lax.GatherDimensionNumbers(
    offset_dims=(), collapsed_slice_dims=(0,), start_index_map=(0,))


def _bcast_lane(vec, j):
    idx = jnp.full((L, 1), j, jnp.int32)
    return lax.gather(vec, idx, _BCAST_DNUMS, (1,),
                      mode=lax.GatherScatterMode.PROMISE_IN_BOUNDS)


@functools.partial(
    pl.kernel,
    mesh=_mesh,
    compiler_params=pltpu.CompilerParams(use_tc_tiling_on_sc=False),
    out_type=jax.ShapeDtypeStruct((NC, OUT_F, B), jnp.float32),
    scratch_types=[
        pltpu.VMEM((CPT,), jnp.int32),
        pltpu.VMEM((NBLK, K), jnp.float32),
        pltpu.VMEM((2, K), jnp.int32),
        pltpu.VMEM((2, K, B), jnp.float32),
        pltpu.VMEM((2, K, B), jnp.float32),
        pltpu.VMEM_SHARED((OUT_F, B), jnp.float32),
        pltpu.SemaphoreType.DMA,
        pltpu.SemaphoreType.DMA,
        pltpu.SemaphoreType.DMA,
        pltpu.SemaphoreType.DMA,
        pltpu.SemaphoreType.DMA,
        pltpu.SemaphoreType.DMA,
    ],
)
def _sc_spmm(xt_hbm, cols_hbm, rows_hbm, w_hbm, out_hbm,
             cols_all, w_all, rows_v, gbuf, sbuf, acc,
             gsem0, gsem1, ssem0, ssem1, rsem0, rsem1):
    gsem = (gsem0, gsem1)
    ssem = (ssem0, ssem1)
    rsem = (rsem0, rsem1)
    cid = lax.axis_index("c")
    sid = lax.axis_index("s")
    wid = sid * NC + cid
    base = wid * CPT

    zero16 = jnp.zeros((L,), jnp.float32)

    def zbody(i, _):
        gbuf[0, i // 4, pl.ds((i % 4) * L, L)] = zero16
        return 0

    lax.fori_loop(0, K * 4, zbody, 0)
    rpt = OUT_F // NS
    for j in range(rpt // K):
        pltpu.sync_copy(gbuf.at[0], acc.at[pl.ds(sid * rpt + j * K, K)])

    pltpu.sync_copy(cols_hbm.at[pl.ds(base, CPT)], cols_all)
    pltpu.sync_copy(w_hbm.at[wid], w_all)
    plsc.subcore_barrier()

    for j in range(2):
        pltpu.async_copy(xt_hbm.at[cols_all.at[pl.ds(j * K, K)]],
                         gbuf.at[j], gsem[j])

    def outer(g, _):
        for j in range(2):
            b = g * 2 + j

            @pl.when(g >= 1)
            def _():
                pltpu.make_async_copy(
                    sbuf.at[j], acc.at[rows_v.at[j]], ssem[j]).wait()

            pltpu.async_copy(rows_hbm.at[wid, b], rows_v.at[j], rsem[j])
            pltpu.make_async_copy(
                xt_hbm.at[cols_all.at[pl.ds(b * K, K)]],
                gbuf.at[j], gsem[j]).wait()

            def mul16(i16, _):
                w16 = w_all[b, pl.ds(i16 * L, L)]
                for jj in range(L):
                    wb = _bcast_lane(w16, jj)
                    e = i16 * L + jj
                    for v in range(B // L):
                        sbuf[j, e, pl.ds(v * L, L)] = (
                            gbuf[j, e, pl.ds(v * L, L)] * wb)
                return 0

            lax.fori_loop(0, K // L, mul16, 0)

            @pl.when(g < NBLK // 2 - 1)
            def _():
                pltpu.async_copy(
                    xt_hbm.at[cols_all.at[pl.ds((b + 2) * K, K)]],
                    gbuf.at[j], gsem[j])

            pltpu.make_async_copy(rows_hbm.at[wid, b], rows_v.at[j],
                                  rsem[j]).wait()
            pltpu.async_copy(sbuf.at[j], acc.at[rows_v.at[j]], ssem[j],
                             add=True)
        return 0

    lax.fori_loop(0, NBLK // 2, outer, 0)

    for j in range(2):
        pltpu.make_async_copy(sbuf.at[j], acc.at[rows_v.at[j]],
                              ssem[j]).wait()
    plsc.subcore_barrier()

    pltpu.sync_copy(acc.at[pl.ds(sid * rpt, rpt)],
                    out_hbm.at[cid, pl.ds(sid * rpt, rpt)])


_RB = 1024


def _combine_body(p_ref, b_ref, o_ref):
    s = p_ref[0] + p_ref[1]
    o_ref[...] = s.T + b_ref[...]


def kernel(inputs, connectivity, weights, bias):
    lead = inputs.shape[:-1]
    x = inputs.reshape(-1, inputs.shape[-1])
    xt = x.T

    rows = connectivity[0]
    cols = connectivity[1]
    pad = NNZ_PAD - NNZ
    cols_p = jnp.concatenate([cols, jnp.zeros((pad,), jnp.int32)])
    rows_p = jnp.concatenate([rows, jnp.zeros((pad,), jnp.int32)])
    w_p = jnp.concatenate([weights, jnp.zeros((pad,), jnp.float32)])
    rows_3d = rows_p.reshape(NW, NBLK, K)
    w_3d = w_p.reshape(NW, NBLK, K)

    partial = _sc_spmm(xt, cols_p, rows_3d, w_3d)

    out = pl.pallas_call(
        _combine_body,
        grid=(OUT_F // _RB,),
        in_specs=[
            pl.BlockSpec((NC, _RB, B), lambda i: (0, i, 0)),
            pl.BlockSpec((1, _RB), lambda i: (0, i)),
        ],
        out_specs=pl.BlockSpec((B, _RB), lambda i: (0, i)),
        out_shape=jax.ShapeDtypeStruct((B, OUT_F), jnp.float32),
    )(partial, bias.reshape(1, OUT_F))
    return out.reshape((*lead, OUT_F))
